# baseline (speedup 1.0000x reference)
import functools

import jax
import jax.numpy as jnp
from jax import lax
from jax.experimental import pallas as pl
from jax.experimental.pallas import tpu as pltpu
from jax.experimental.pallas import tpu_sc as plsc

NC = 2
NS = 16
NW = NC * NS
CB = 80


def _gelu(x):
    return 0.5 * x * (1.0 + lax.erf(x * 0.7071067811865475))


def _iota16():
    return lax.iota(jnp.int32, 16)


def _rsqrt(x):
    xi = plsc.bitcast(x, jnp.int32)
    yi = jnp.int32(0x5F3759DF) - lax.shift_right_logical(xi, 1)
    y = plsc.bitcast(yi, jnp.float32)
    hx = x * 0.5
    for _ in range(3):
        y = y * (1.5 - hx * y * y)
    return y


def _edge_geom_kernel(E, N):
    epw = E // NW
    nchunk = epw // CB
    mesh = plsc.VectorSubcoreMesh(core_axis_name="c", subcore_axis_name="s")

    @functools.partial(
        pl.kernel,
        mesh=mesh,
        out_type=jax.ShapeDtypeStruct((E * 4,), jnp.float32),
        compiler_params=pltpu.CompilerParams(needs_layout_passes=False),
        scratch_types=[
            pltpu.VMEM((N,), jnp.float32),
            pltpu.VMEM((N,), jnp.float32),
            pltpu.VMEM((N,), jnp.float32),
            pltpu.VMEM((CB,), jnp.int32),
            pltpu.VMEM((CB,), jnp.int32),
            pltpu.VMEM((CB * 4,), jnp.float32),
        ],
    )
    def k(px_hbm, py_hbm, pz_hbm, src_hbm, dst_hbm, out_hbm,
          px_v, py_v, pz_v, src_v, dst_v, o_v):
        wid = lax.axis_index("s") * NC + lax.axis_index("c")
        base0 = wid * epw
        pltpu.sync_copy(px_hbm, px_v)
        pltpu.sync_copy(py_hbm, py_v)
        pltpu.sync_copy(pz_hbm, pz_v)
        it = _iota16()

        def chunk(g, _):
            base = base0 + g * CB
            pltpu.sync_copy(src_hbm.at[pl.ds(base, CB)], src_v)
            pltpu.sync_copy(dst_hbm.at[pl.ds(base, CB)], dst_v)

            def grp(t, _):
                si = src_v[pl.ds(t * 16, 16)]
                di = dst_v[pl.ds(t * 16, 16)]
                dx = plsc.load_gather(px_v, [di]) - plsc.load_gather(px_v, [si])
                dy = plsc.load_gather(py_v, [di]) - plsc.load_gather(py_v, [si])
                dz = plsc.load_gather(pz_v, [di]) - plsc.load_gather(pz_v, [si])
                r2 = jnp.maximum(dx * dx + dy * dy + dz * dz, 1e-24)
                dist = r2 * _rsqrt(r2)
                inv = 1.0 / (dist + 1e-6)
                fo = (it + t * 16) * 4
                plsc.store_scatter(o_v, [fo], dist)
                plsc.store_scatter(o_v, [fo + 1], dx * inv)
                plsc.store_scatter(o_v, [fo + 2], dy * inv)
                plsc.store_scatter(o_v, [fo + 3], dz * inv)
                return 0

            lax.fori_loop(0, CB // 16, grp, 0)
            pltpu.sync_copy(o_v, out_hbm.at[pl.ds(base * 4, CB * 4)])
            return 0

        lax.fori_loop(0, nchunk, chunk, 0)

    return k


def _msg_scatter_kernel(E, N, cb):
    epc = E // NC
    eps = epc // NS
    nchunk = eps // cb
    assert eps % cb == 0
    npair = nchunk // 2
    tail = nchunk % 2
    zrows = (N // NS) // 8 * 8
    rem = N - zrows * NS
    mesh = plsc.VectorSubcoreMesh(core_axis_name="c", subcore_axis_name="s")

    @functools.partial(
        pl.kernel,
        mesh=mesh,
        out_type=jax.ShapeDtypeStruct((NC, N, 128), jnp.float32),
        compiler_params=pltpu.CompilerParams(needs_layout_passes=False),
        scratch_types=[
            pltpu.VMEM((cb,), jnp.int32),
            pltpu.VMEM((cb,), jnp.int32),
            pltpu.VMEM((cb, 128), jnp.float32),
            pltpu.VMEM((cb, 128), jnp.float32),
            pltpu.VMEM((cb,), jnp.int32),
            pltpu.VMEM((cb,), jnp.int32),
            pltpu.VMEM((cb, 128), jnp.float32),
            pltpu.VMEM((cb, 128), jnp.float32),
            pltpu.VMEM_SHARED((N, 128), jnp.float32),
            pltpu.SemaphoreType.DMA,
            pltpu.SemaphoreType.DMA,
            pltpu.SemaphoreType.DMA,
            pltpu.SemaphoreType.DMA,
        ],
    )
    def k(nf_hbm, er_hbm, src_hbm, dst_hbm, zero_hbm, out_hbm,
          si_a, di_a, nf_a, er_a, si_b, di_b, nf_b, er_b,
          agg_sh, sna, sea, snb, seb):
        cid = lax.axis_index("c")
        sid = lax.axis_index("s")
        base0 = cid * epc + sid * eps

        pltpu.sync_copy(zero_hbm, agg_sh.at[pl.ds(sid * zrows, zrows)])
        if rem:
            @pl.when(sid == NS - 1)
            def _():
                pltpu.sync_copy(zero_hbm.at[pl.ds(0, rem)],
                                agg_sh.at[pl.ds(NS * zrows, rem)])
        plsc.subcore_barrier()

        def start(g, si_v, di_v, nf_v, er_v, snf, ser):
            base = base0 + g * cb
            pltpu.sync_copy(src_hbm.at[pl.ds(base, cb)], si_v)
            pltpu.sync_copy(dst_hbm.at[pl.ds(base, cb)], di_v)
            cp1 = pltpu.async_copy(nf_hbm.at[si_v], nf_v, snf)
            cp2 = pltpu.async_copy(er_hbm.at[pl.ds(base, cb)], er_v, ser)
            return cp1, cp2

        def finish(cps, di_v, nf_v, er_v):
            cp1, cp2 = cps
            cp1.wait()
            cp2.wait()

            def mul(e, _):
                for j in range(8):
                    s = pl.ds(j * 16, 16)
                    nf_v[e, s] = nf_v[e, s] * er_v[e, s]
                return 0

            lax.fori_loop(0, cb, mul, 0)
            pltpu.sync_copy(nf_v, agg_sh.at[di_v], add=True)

        def pair(h, _):
            cps_a = start(2 * h, si_a, di_a, nf_a, er_a, sna, sea)
            cps_b = start(2 * h + 1, si_b, di_b, nf_b, er_b, snb, seb)
            finish(cps_a, di_a, nf_a, er_a)
            finish(cps_b, di_b, nf_b, er_b)
            return 0

        lax.fori_loop(0, npair, pair, 0)
        if tail:
            cps = start(nchunk - 1, si_a, di_a, nf_a, er_a, sna, sea)
            finish(cps, di_a, nf_a, er_a)
        plsc.subcore_barrier()
        pltpu.sync_copy(agg_sh.at[pl.ds(sid * zrows, zrows)],
                        out_hbm.at[cid, pl.ds(sid * zrows, zrows)])
        if rem:
            @pl.when(sid == NS - 1)
            def _():
                pltpu.sync_copy(agg_sh.at[pl.ds(NS * zrows, rem)],
                                out_hbm.at[cid, pl.ds(NS * zrows, rem)])

    return k


def _edge_mlp(edge_in, We, be):
    E = edge_in.shape[0]
    BE = 2000
    grid = E // BE

    def body(x_ref, we_ref, be_ref, o_ref):
        h = jnp.dot(x_ref[...], we_ref[...],
                    preferred_element_type=jnp.float32) + be_ref[...]
        o_ref[...] = _gelu(h)

    return pl.pallas_call(
        body,
        grid=(grid,),
        in_specs=[
            pl.BlockSpec((BE, 4), lambda i: (i, 0)),
            pl.BlockSpec((4, 128), lambda i: (0, 0)),
            pl.BlockSpec((1, 128), lambda i: (0, 0)),
        ],
        out_specs=pl.BlockSpec((BE, 128), lambda i: (i, 0)),
        out_shape=jax.ShapeDtypeStruct((E, 128), jnp.float32),
    )(edge_in, We, be)


def _node_mlp(partials, Wn, bn):
    N = partials[0].shape[1]
    BN = 1000
    grid = N // BN

    def body(p_ref, q_ref, wn_ref, bn_ref, o_ref):
        agg = (p_ref[0] + p_ref[1]) + (q_ref[0] + q_ref[1])
        h = jnp.dot(agg, wn_ref[...],
                    preferred_element_type=jnp.float32) + bn_ref[...]
        o_ref[...] = _gelu(h)

    return pl.pallas_call(
        body,
        grid=(grid,),
        in_specs=[
            pl.BlockSpec((2, BN, 128), lambda i: (0, i, 0)),
            pl.BlockSpec((2, BN, 128), lambda i: (0, i, 0)),
            pl.BlockSpec((128, 128), lambda i: (0, 0)),
            pl.BlockSpec((1, 128), lambda i: (0, 0)),
        ],
        out_specs=pl.BlockSpec((BN, 128), lambda i: (i, 0)),
        out_shape=jax.ShapeDtypeStruct((N, 128), jnp.float32),
    )(*partials, Wn, bn)


def kernel(node_features, positions, edge_index, We, be, Wn, bn):
    N, D = node_features.shape
    E = edge_index.shape[0]
    ei = edge_index.astype(jnp.int32)
    src = ei[:, 0]
    dst = ei[:, 1]
    px = positions[:, 0]
    py = positions[:, 1]
    pz = positions[:, 2]

    edge_in = _edge_geom_kernel(E, N)(px, py, pz, src, dst).reshape(E, 4)

    Eh = E // 2
    zrows = (N // NS) // 8 * 8
    zero = jnp.zeros((zrows, 128), jnp.float32)
    scatter = _msg_scatter_kernel(Eh, N, 40)
    partials = []
    for h in range(2):
        sl = slice(h * Eh, (h + 1) * Eh)
        er_h = _edge_mlp(edge_in[sl], We, be.reshape(1, 128))
        partials.append(scatter(node_features, er_h, src[sl], dst[sl], zero))
    return _node_mlp(partials, Wn, bn.reshape(1, 128))


# speedup vs baseline: 1.3012x; 1.3012x over previous
import functools

import jax
import jax.numpy as jnp
from jax import lax
from jax.experimental import pallas as pl
from jax.experimental.pallas import tpu as pltpu
from jax.experimental.pallas import tpu_sc as plsc

NC = 2
NS = 16
NW = NC * NS
CB = 80


def _gelu(x):
    return 0.5 * x * (1.0 + lax.erf(x * 0.7071067811865475))


def _iota16():
    return lax.iota(jnp.int32, 16)


def _rsqrt(x):
    xi = plsc.bitcast(x, jnp.int32)
    yi = jnp.int32(0x5F3759DF) - lax.shift_right_logical(xi, 1)
    y = plsc.bitcast(yi, jnp.float32)
    hx = x * 0.5
    for _ in range(3):
        y = y * (1.5 - hx * y * y)
    return y


def _edge_geom_kernel(E, N):
    epw = E // NW
    nchunk = epw // CB
    mesh = plsc.VectorSubcoreMesh(core_axis_name="c", subcore_axis_name="s")

    @functools.partial(
        pl.kernel,
        mesh=mesh,
        out_type=jax.ShapeDtypeStruct((E * 4,), jnp.float32),
        compiler_params=pltpu.CompilerParams(needs_layout_passes=False),
        scratch_types=[
            pltpu.VMEM((N,), jnp.float32),
            pltpu.VMEM((N,), jnp.float32),
            pltpu.VMEM((N,), jnp.float32),
            pltpu.VMEM((E // NW,), jnp.int32),
            pltpu.VMEM((E // NW,), jnp.int32),
            pltpu.VMEM((CB * 4,), jnp.float32),
        ],
    )
    def k(px_hbm, py_hbm, pz_hbm, src_hbm, dst_hbm, out_hbm,
          px_v, py_v, pz_v, src_v, dst_v, o_v):
        wid = lax.axis_index("s") * NC + lax.axis_index("c")
        base0 = wid * epw
        pltpu.sync_copy(px_hbm, px_v)
        pltpu.sync_copy(py_hbm, py_v)
        pltpu.sync_copy(pz_hbm, pz_v)
        pltpu.sync_copy(src_hbm.at[pl.ds(base0, epw)], src_v)
        pltpu.sync_copy(dst_hbm.at[pl.ds(base0, epw)], dst_v)
        it = _iota16()

        def chunk(g, _):
            base = base0 + g * CB

            def grp(t, _):
                si = src_v[pl.ds(g * CB + t * 16, 16)]
                di = dst_v[pl.ds(g * CB + t * 16, 16)]
                dx = plsc.load_gather(px_v, [di]) - plsc.load_gather(px_v, [si])
                dy = plsc.load_gather(py_v, [di]) - plsc.load_gather(py_v, [si])
                dz = plsc.load_gather(pz_v, [di]) - plsc.load_gather(pz_v, [si])
                r2 = jnp.maximum(dx * dx + dy * dy + dz * dz, 1e-24)
                dist = r2 * _rsqrt(r2)
                inv = 1.0 / (dist + 1e-6)
                fo = (it + t * 16) * 4
                plsc.store_scatter(o_v, [fo], dist)
                plsc.store_scatter(o_v, [fo + 1], dx * inv)
                plsc.store_scatter(o_v, [fo + 2], dy * inv)
                plsc.store_scatter(o_v, [fo + 3], dz * inv)
                return 0

            lax.fori_loop(0, CB // 16, grp, 0)
            pltpu.sync_copy(o_v, out_hbm.at[pl.ds(base * 4, CB * 4)])
            return 0

        lax.fori_loop(0, nchunk, chunk, 0)

    return k


def _msg_scatter_kernel(E, N, cb):
    epc = E // NC
    eps = epc // NS
    nchunk = eps // cb
    assert eps % cb == 0
    npair = nchunk // 2
    tail = nchunk % 2
    zrows = (N // NS) // 8 * 8
    rem = N - zrows * NS
    mesh = plsc.VectorSubcoreMesh(core_axis_name="c", subcore_axis_name="s")

    @functools.partial(
        pl.kernel,
        mesh=mesh,
        out_type=jax.ShapeDtypeStruct((NC, N, 128), jnp.float32),
        compiler_params=pltpu.CompilerParams(needs_layout_passes=False),
        scratch_types=[
            pltpu.VMEM((cb,), jnp.int32),
            pltpu.VMEM((cb,), jnp.int32),
            pltpu.VMEM((cb, 128), jnp.float32),
            pltpu.VMEM((cb, 128), jnp.float32),
            pltpu.VMEM((cb,), jnp.int32),
            pltpu.VMEM((cb,), jnp.int32),
            pltpu.VMEM((cb, 128), jnp.float32),
            pltpu.VMEM((cb, 128), jnp.float32),
            pltpu.VMEM_SHARED((N, 128), jnp.float32),
            pltpu.SemaphoreType.DMA,
            pltpu.SemaphoreType.DMA,
            pltpu.SemaphoreType.DMA,
            pltpu.SemaphoreType.DMA,
            pltpu.SemaphoreType.DMA,
            pltpu.SemaphoreType.DMA,
            pltpu.SemaphoreType.DMA,
            pltpu.SemaphoreType.DMA,
        ],
    )
    def k(nf_hbm, er_hbm, src_hbm, dst_hbm, zero_hbm, out_hbm,
          si_a, di_a, nf_a, er_a, si_b, di_b, nf_b, er_b,
          agg_sh, sna, sea, snb, seb, ssa, sda, ssb, sdb):
        cid = lax.axis_index("c")
        sid = lax.axis_index("s")
        base0 = cid * epc + sid * eps

        pltpu.sync_copy(zero_hbm, agg_sh.at[pl.ds(sid * zrows, zrows)])
        if rem:
            @pl.when(sid == NS - 1)
            def _():
                pltpu.sync_copy(zero_hbm.at[pl.ds(0, rem)],
                                agg_sh.at[pl.ds(NS * zrows, rem)])
        plsc.subcore_barrier()

        def start(g, si_v, di_v, nf_v, er_v, ssi, sdi, snf, ser):
            base = base0 + g * cb
            pltpu.sync_copy(src_hbm.at[pl.ds(base, cb)], si_v)
            pltpu.sync_copy(dst_hbm.at[pl.ds(base, cb)], di_v)
            cp1 = pltpu.async_copy(nf_hbm.at[si_v], nf_v, snf)
            cp2 = pltpu.async_copy(er_hbm.at[pl.ds(base, cb)], er_v, ser)
            return cp1, cp2

        def finish(cps, di_v, nf_v, er_v):
            cp1, cp2 = cps
            cp1.wait()
            cp2.wait()

            def mul(e, _):
                for j in range(8):
                    s = pl.ds(j * 16, 16)
                    nf_v[e, s] = nf_v[e, s] * er_v[e, s]
                return 0

            lax.fori_loop(0, cb, mul, 0)
            pltpu.sync_copy(nf_v, agg_sh.at[di_v], add=True)

        def pair(h, _):
            cps_a = start(2 * h, si_a, di_a, nf_a, er_a, ssa, sda, sna, sea)
            cps_b = start(2 * h + 1, si_b, di_b, nf_b, er_b, ssb, sdb,
                          snb, seb)
            finish(cps_a, di_a, nf_a, er_a)
            finish(cps_b, di_b, nf_b, er_b)
            return 0

        lax.fori_loop(0, npair, pair, 0)
        if tail:
            cps = start(nchunk - 1, si_a, di_a, nf_a, er_a, ssa, sda,
                        sna, sea)
            finish(cps, di_a, nf_a, er_a)
        plsc.subcore_barrier()
        pltpu.sync_copy(agg_sh.at[pl.ds(sid * zrows, zrows)],
                        out_hbm.at[cid, pl.ds(sid * zrows, zrows)])
        if rem:
            @pl.when(sid == NS - 1)
            def _():
                pltpu.sync_copy(agg_sh.at[pl.ds(NS * zrows, rem)],
                                out_hbm.at[cid, pl.ds(NS * zrows, rem)])

    return k


def _edge_mlp(edge_in, We, be):
    E = edge_in.shape[0]
    BE = 2000
    grid = E // BE

    def body(x_ref, we_ref, be_ref, o_ref):
        h = jnp.dot(x_ref[...], we_ref[...],
                    preferred_element_type=jnp.float32) + be_ref[...]
        o_ref[...] = _gelu(h)

    return pl.pallas_call(
        body,
        grid=(grid,),
        in_specs=[
            pl.BlockSpec((BE, 4), lambda i: (i, 0)),
            pl.BlockSpec((4, 128), lambda i: (0, 0)),
            pl.BlockSpec((1, 128), lambda i: (0, 0)),
        ],
        out_specs=pl.BlockSpec((BE, 128), lambda i: (i, 0)),
        out_shape=jax.ShapeDtypeStruct((E, 128), jnp.float32),
    )(edge_in, We, be)


def _node_mlp(partials, Wn, bn):
    N = partials[0].shape[1]
    BN = 1000
    grid = N // BN

    np_ = len(partials)

    def body(*refs):
        p_refs, (wn_ref, bn_ref, o_ref) = refs[:np_], refs[np_:]
        agg = sum(p[0] + p[1] for p in p_refs)
        h = jnp.dot(agg, wn_ref[...],
                    preferred_element_type=jnp.float32) + bn_ref[...]
        o_ref[...] = _gelu(h)

    return pl.pallas_call(
        body,
        grid=(grid,),
        in_specs=[pl.BlockSpec((2, BN, 128), lambda i: (0, i, 0))
                  for _ in range(np_)] + [
            pl.BlockSpec((128, 128), lambda i: (0, 0)),
            pl.BlockSpec((1, 128), lambda i: (0, 0)),
        ],
        out_specs=pl.BlockSpec((BN, 128), lambda i: (i, 0)),
        out_shape=jax.ShapeDtypeStruct((N, 128), jnp.float32),
    )(*partials, Wn, bn)


def kernel(node_features, positions, edge_index, We, be, Wn, bn):
    N, D = node_features.shape
    E = edge_index.shape[0]
    ei = edge_index.astype(jnp.int32)
    src = ei[:, 0]
    dst = ei[:, 1]
    px = positions[:, 0]
    py = positions[:, 1]
    pz = positions[:, 2]

    edge_in = _edge_geom_kernel(E, N)(px, py, pz, src, dst).reshape(E, 4)
    edge_repr = _edge_mlp(edge_in, We, be.reshape(1, 128))

    zrows = (N // NS) // 8 * 8
    zero = jnp.zeros((zrows, 128), jnp.float32)
    partials = _msg_scatter_kernel(E, N, CB)(node_features, edge_repr,
                                             src, dst, zero)
    return _node_mlp([partials], Wn, bn.reshape(1, 128))


# speedup vs baseline: 1.3116x; 1.0080x over previous
import functools

import jax
import jax.numpy as jnp
from jax import lax
from jax.experimental import pallas as pl
from jax.experimental.pallas import tpu as pltpu
from jax.experimental.pallas import tpu_sc as plsc

NC = 2
NS = 16
NW = NC * NS
CB = 80


def _gelu(x):
    return 0.5 * x * (1.0 + lax.erf(x * 0.7071067811865475))


def _iota16():
    return lax.iota(jnp.int32, 16)


def _rsqrt(x):
    xi = plsc.bitcast(x, jnp.int32)
    yi = jnp.int32(0x5F3759DF) - lax.shift_right_logical(xi, 1)
    y = plsc.bitcast(yi, jnp.float32)
    hx = x * 0.5
    for _ in range(3):
        y = y * (1.5 - hx * y * y)
    return y


def _edge_geom_kernel(E, N):
    epw = E // NW
    nchunk = epw // CB
    mesh = plsc.VectorSubcoreMesh(core_axis_name="c", subcore_axis_name="s")

    @functools.partial(
        pl.kernel,
        mesh=mesh,
        out_type=jax.ShapeDtypeStruct((E * 4,), jnp.float32),
        compiler_params=pltpu.CompilerParams(needs_layout_passes=False),
        scratch_types=[
            pltpu.VMEM((N,), jnp.float32),
            pltpu.VMEM((N,), jnp.float32),
            pltpu.VMEM((N,), jnp.float32),
            pltpu.VMEM((E // NW,), jnp.int32),
            pltpu.VMEM((E // NW,), jnp.int32),
            pltpu.VMEM((CB * 4,), jnp.float32),
        ],
    )
    def k(px_hbm, py_hbm, pz_hbm, src_hbm, dst_hbm, out_hbm,
          px_v, py_v, pz_v, src_v, dst_v, o_v):
        wid = lax.axis_index("s") * NC + lax.axis_index("c")
        base0 = wid * epw
        pltpu.sync_copy(px_hbm, px_v)
        pltpu.sync_copy(py_hbm, py_v)
        pltpu.sync_copy(pz_hbm, pz_v)
        pltpu.sync_copy(src_hbm.at[pl.ds(base0, epw)], src_v)
        pltpu.sync_copy(dst_hbm.at[pl.ds(base0, epw)], dst_v)
        it = _iota16()

        def chunk(g, _):
            base = base0 + g * CB

            def grp(t, _):
                si = src_v[pl.ds(g * CB + t * 16, 16)]
                di = dst_v[pl.ds(g * CB + t * 16, 16)]
                dx = plsc.load_gather(px_v, [di]) - plsc.load_gather(px_v, [si])
                dy = plsc.load_gather(py_v, [di]) - plsc.load_gather(py_v, [si])
                dz = plsc.load_gather(pz_v, [di]) - plsc.load_gather(pz_v, [si])
                r2 = jnp.maximum(dx * dx + dy * dy + dz * dz, 1e-24)
                dist = r2 * _rsqrt(r2)
                inv = 1.0 / (dist + 1e-6)
                fo = (it + t * 16) * 4
                plsc.store_scatter(o_v, [fo], dist)
                plsc.store_scatter(o_v, [fo + 1], dx * inv)
                plsc.store_scatter(o_v, [fo + 2], dy * inv)
                plsc.store_scatter(o_v, [fo + 3], dz * inv)
                return 0

            lax.fori_loop(0, CB // 16, grp, 0)
            pltpu.sync_copy(o_v, out_hbm.at[pl.ds(base * 4, CB * 4)])
            return 0

        lax.fori_loop(0, nchunk, chunk, 0)

    return k


def _msg_scatter_kernel(E, N, cb):
    epc = E // NC
    eps = epc // NS
    nchunk = eps // cb
    assert eps % cb == 0
    npair = nchunk // 2
    tail = nchunk % 2
    zrows = (N // NS) // 8 * 8
    rem = N - zrows * NS
    mesh = plsc.VectorSubcoreMesh(core_axis_name="c", subcore_axis_name="s")

    @functools.partial(
        pl.kernel,
        mesh=mesh,
        out_type=jax.ShapeDtypeStruct((NC, N, 128), jnp.float32),
        compiler_params=pltpu.CompilerParams(needs_layout_passes=False),
        scratch_types=[
            pltpu.VMEM((cb,), jnp.int32),
            pltpu.VMEM((cb,), jnp.int32),
            pltpu.VMEM((cb, 128), jnp.float32),
            pltpu.VMEM((cb, 128), jnp.float32),
            pltpu.VMEM((cb,), jnp.int32),
            pltpu.VMEM((cb,), jnp.int32),
            pltpu.VMEM((cb, 128), jnp.float32),
            pltpu.VMEM((cb, 128), jnp.float32),
            pltpu.VMEM_SHARED((N, 128), jnp.float32),
            pltpu.SemaphoreType.DMA,
            pltpu.SemaphoreType.DMA,
            pltpu.SemaphoreType.DMA,
            pltpu.SemaphoreType.DMA,
            pltpu.SemaphoreType.DMA,
            pltpu.SemaphoreType.DMA,
            pltpu.SemaphoreType.DMA,
            pltpu.SemaphoreType.DMA,
        ],
    )
    def k(nf_hbm, er_hbm, src_hbm, dst_hbm, zero_hbm, out_hbm,
          si_a, di_a, nf_a, er_a, si_b, di_b, nf_b, er_b,
          agg_sh, sna, sea, snb, seb, ssa, sda, ssb, sdb):
        cid = lax.axis_index("c")
        sid = lax.axis_index("s")
        base0 = cid * epc + sid * eps

        pltpu.sync_copy(zero_hbm, agg_sh.at[pl.ds(sid * zrows, zrows)])
        if rem:
            @pl.when(sid == NS - 1)
            def _():
                pltpu.sync_copy(zero_hbm.at[pl.ds(0, rem)],
                                agg_sh.at[pl.ds(NS * zrows, rem)])
        plsc.subcore_barrier()

        def start(g, si_v, di_v, nf_v, er_v, ssi, sdi, snf, ser):
            base = base0 + g * cb
            pltpu.sync_copy(src_hbm.at[pl.ds(base, cb)], si_v)
            c_di = pltpu.async_copy(dst_hbm.at[pl.ds(base, cb)], di_v, sdi)
            cp1 = pltpu.async_copy(nf_hbm.at[si_v], nf_v, snf)
            cp2 = pltpu.async_copy(er_hbm.at[pl.ds(base, cb)], er_v, ser)
            return cp1, cp2, c_di

        def finish(cps, di_v, nf_v, er_v):
            cp1, cp2, c_di = cps
            cp1.wait()
            cp2.wait()

            def mul(e, _):
                for j in range(8):
                    s = pl.ds(j * 16, 16)
                    nf_v[e, s] = nf_v[e, s] * er_v[e, s]
                return 0

            lax.fori_loop(0, cb, mul, 0)
            c_di.wait()
            pltpu.sync_copy(nf_v, agg_sh.at[di_v], add=True)

        def pair(h, _):
            cps_a = start(2 * h, si_a, di_a, nf_a, er_a, ssa, sda, sna, sea)
            cps_b = start(2 * h + 1, si_b, di_b, nf_b, er_b, ssb, sdb,
                          snb, seb)
            finish(cps_a, di_a, nf_a, er_a)
            finish(cps_b, di_b, nf_b, er_b)
            return 0

        lax.fori_loop(0, npair, pair, 0)
        if tail:
            cps = start(nchunk - 1, si_a, di_a, nf_a, er_a, ssa, sda,
                        sna, sea)
            finish(cps, di_a, nf_a, er_a)
        plsc.subcore_barrier()
        pltpu.sync_copy(agg_sh.at[pl.ds(sid * zrows, zrows)],
                        out_hbm.at[cid, pl.ds(sid * zrows, zrows)])
        if rem:
            @pl.when(sid == NS - 1)
            def _():
                pltpu.sync_copy(agg_sh.at[pl.ds(NS * zrows, rem)],
                                out_hbm.at[cid, pl.ds(NS * zrows, rem)])

    return k


def _edge_mlp(edge_in, We, be):
    E = edge_in.shape[0]
    BE = 2000
    grid = E // BE

    def body(x_ref, we_ref, be_ref, o_ref):
        h = jnp.dot(x_ref[...], we_ref[...],
                    preferred_element_type=jnp.float32) + be_ref[...]
        o_ref[...] = _gelu(h)

    return pl.pallas_call(
        body,
        grid=(grid,),
        in_specs=[
            pl.BlockSpec((BE, 4), lambda i: (i, 0)),
            pl.BlockSpec((4, 128), lambda i: (0, 0)),
            pl.BlockSpec((1, 128), lambda i: (0, 0)),
        ],
        out_specs=pl.BlockSpec((BE, 128), lambda i: (i, 0)),
        out_shape=jax.ShapeDtypeStruct((E, 128), jnp.float32),
    )(edge_in, We, be)


def _node_mlp(partials, Wn, bn):
    N = partials[0].shape[1]
    BN = 1000
    grid = N // BN

    np_ = len(partials)

    def body(*refs):
        p_refs, (wn_ref, bn_ref, o_ref) = refs[:np_], refs[np_:]
        agg = sum(p[0] + p[1] for p in p_refs)
        h = jnp.dot(agg, wn_ref[...],
                    preferred_element_type=jnp.float32) + bn_ref[...]
        o_ref[...] = _gelu(h)

    return pl.pallas_call(
        body,
        grid=(grid,),
        in_specs=[pl.BlockSpec((2, BN, 128), lambda i: (0, i, 0))
                  for _ in range(np_)] + [
            pl.BlockSpec((128, 128), lambda i: (0, 0)),
            pl.BlockSpec((1, 128), lambda i: (0, 0)),
        ],
        out_specs=pl.BlockSpec((BN, 128), lambda i: (i, 0)),
        out_shape=jax.ShapeDtypeStruct((N, 128), jnp.float32),
    )(*partials, Wn, bn)


def kernel(node_features, positions, edge_index, We, be, Wn, bn):
    N, D = node_features.shape
    E = edge_index.shape[0]
    ei = edge_index.astype(jnp.int32)
    src = ei[:, 0]
    dst = ei[:, 1]
    px = positions[:, 0]
    py = positions[:, 1]
    pz = positions[:, 2]

    edge_in = _edge_geom_kernel(E, N)(px, py, pz, src, dst).reshape(E, 4)
    edge_repr = _edge_mlp(edge_in, We, be.reshape(1, 128))

    zrows = (N // NS) // 8 * 8
    zero = jnp.zeros((zrows, 128), jnp.float32)
    partials = _msg_scatter_kernel(E, N, CB)(node_features, edge_repr,
                                             src, dst, zero)
    return _node_mlp([partials], Wn, bn.reshape(1, 128))


# speedup vs baseline: 1.4532x; 1.1080x over previous
import functools

import jax
import jax.numpy as jnp
from jax import lax
from jax.experimental import pallas as pl
from jax.experimental.pallas import tpu as pltpu
from jax.experimental.pallas import tpu_sc as plsc

NC = 2
NS = 16
NW = NC * NS
CB = 80


def _gelu(x):
    return 0.5 * x * (1.0 + lax.erf(x * 0.7071067811865475))


def _iota16():
    return lax.iota(jnp.int32, 16)


def _rsqrt(x):
    xi = plsc.bitcast(x, jnp.int32)
    yi = jnp.int32(0x5F3759DF) - lax.shift_right_logical(xi, 1)
    y = plsc.bitcast(yi, jnp.float32)
    hx = x * 0.5
    for _ in range(3):
        y = y * (1.5 - hx * y * y)
    return y


def _edge_geom_kernel(E, N):
    epw = E // NW
    nchunk = epw // CB
    mesh = plsc.VectorSubcoreMesh(core_axis_name="c", subcore_axis_name="s")

    @functools.partial(
        pl.kernel,
        mesh=mesh,
        out_type=jax.ShapeDtypeStruct((E * 4,), jnp.float32),
        compiler_params=pltpu.CompilerParams(needs_layout_passes=False),
        scratch_types=[
            pltpu.VMEM((N,), jnp.float32),
            pltpu.VMEM((N,), jnp.float32),
            pltpu.VMEM((N,), jnp.float32),
            pltpu.VMEM((E // NW,), jnp.int32),
            pltpu.VMEM((E // NW,), jnp.int32),
            pltpu.VMEM((CB * 4,), jnp.float32),
        ],
    )
    def k(px_hbm, py_hbm, pz_hbm, src_hbm, dst_hbm, out_hbm,
          px_v, py_v, pz_v, src_v, dst_v, o_v):
        wid = lax.axis_index("s") * NC + lax.axis_index("c")
        base0 = wid * epw
        pltpu.sync_copy(px_hbm, px_v)
        pltpu.sync_copy(py_hbm, py_v)
        pltpu.sync_copy(pz_hbm, pz_v)
        pltpu.sync_copy(src_hbm.at[pl.ds(base0, epw)], src_v)
        pltpu.sync_copy(dst_hbm.at[pl.ds(base0, epw)], dst_v)
        it = _iota16()

        def chunk(g, _):
            base = base0 + g * CB

            def grp(t, _):
                si = src_v[pl.ds(g * CB + t * 16, 16)]
                di = dst_v[pl.ds(g * CB + t * 16, 16)]
                dx = plsc.load_gather(px_v, [di]) - plsc.load_gather(px_v, [si])
                dy = plsc.load_gather(py_v, [di]) - plsc.load_gather(py_v, [si])
                dz = plsc.load_gather(pz_v, [di]) - plsc.load_gather(pz_v, [si])
                r2 = jnp.maximum(dx * dx + dy * dy + dz * dz, 1e-24)
                dist = r2 * _rsqrt(r2)
                inv = 1.0 / (dist + 1e-6)
                fo = (it + t * 16) * 4
                plsc.store_scatter(o_v, [fo], dist)
                plsc.store_scatter(o_v, [fo + 1], dx * inv)
                plsc.store_scatter(o_v, [fo + 2], dy * inv)
                plsc.store_scatter(o_v, [fo + 3], dz * inv)
                return 0

            lax.fori_loop(0, CB // 16, grp, 0)
            pltpu.sync_copy(o_v, out_hbm.at[pl.ds(base * 4, CB * 4)])
            return 0

        lax.fori_loop(0, nchunk, chunk, 0)

    return k


def _msg_scatter_kernel(E, N, cb):
    epc = E // NC
    eps = epc // NS
    nchunk = eps // cb
    assert eps % cb == 0
    npair = nchunk // 2
    tail = nchunk % 2
    zrows = (N // NS) // 8 * 8
    rem = N - zrows * NS
    mesh = plsc.VectorSubcoreMesh(core_axis_name="c", subcore_axis_name="s")

    @functools.partial(
        pl.kernel,
        mesh=mesh,
        out_type=jax.ShapeDtypeStruct((NC, N, 128), jnp.float32),
        compiler_params=pltpu.CompilerParams(needs_layout_passes=False),
        scratch_types=[
            pltpu.VMEM((cb,), jnp.int32),
            pltpu.VMEM((cb,), jnp.int32),
            pltpu.VMEM((cb, 128), jnp.float32),
            pltpu.VMEM((cb, 128), jnp.float32),
            pltpu.VMEM((cb,), jnp.int32),
            pltpu.VMEM((cb,), jnp.int32),
            pltpu.VMEM((cb, 128), jnp.float32),
            pltpu.VMEM((cb, 128), jnp.float32),
            pltpu.VMEM_SHARED((N, 128), jnp.float32),
            pltpu.SemaphoreType.DMA,
            pltpu.SemaphoreType.DMA,
            pltpu.SemaphoreType.DMA,
            pltpu.SemaphoreType.DMA,
            pltpu.SemaphoreType.DMA,
            pltpu.SemaphoreType.DMA,
            pltpu.SemaphoreType.DMA,
            pltpu.SemaphoreType.DMA,
        ],
    )
    def k(nf_hbm, er_hbm, src_hbm, dst_hbm, zero_hbm, out_hbm,
          si_a, di_a, nf_a, er_a, si_b, di_b, nf_b, er_b,
          agg_sh, sna, sea, snb, seb, ssa, sda, ssb, sdb):
        cid = lax.axis_index("c")
        sid = lax.axis_index("s")
        base0 = cid * epc + sid * eps

        pltpu.sync_copy(zero_hbm, agg_sh.at[pl.ds(sid * zrows, zrows)])
        if rem:
            @pl.when(sid == NS - 1)
            def _():
                pltpu.sync_copy(zero_hbm.at[pl.ds(0, rem)],
                                agg_sh.at[pl.ds(NS * zrows, rem)])
        plsc.subcore_barrier()

        def start(g, si_v, di_v, nf_v, er_v, ssi, sdi, snf, ser):
            base = base0 + g * cb
            pltpu.sync_copy(src_hbm.at[pl.ds(base, cb)], si_v)
            c_di = pltpu.async_copy(dst_hbm.at[pl.ds(base, cb)], di_v, sdi)
            cp1 = pltpu.async_copy(nf_hbm.at[si_v], nf_v, snf)
            cp2 = pltpu.async_copy(er_hbm.at[pl.ds(base, cb)], er_v, ser)
            return cp1, cp2, c_di

        def finish(cps, di_v, nf_v, er_v):
            cp1, cp2, c_di = cps
            cp1.wait()
            cp2.wait()

            def mul(e, _):
                for j in range(8):
                    s = pl.ds(j * 16, 16)
                    nf_v[e, s] = nf_v[e, s] * er_v[e, s]
                return 0

            lax.fori_loop(0, cb, mul, 0)
            c_di.wait()
            pltpu.sync_copy(nf_v, agg_sh.at[di_v], add=True)

        def pair(h, _):
            cps_a = start(2 * h, si_a, di_a, nf_a, er_a, ssa, sda, sna, sea)
            cps_b = start(2 * h + 1, si_b, di_b, nf_b, er_b, ssb, sdb,
                          snb, seb)
            finish(cps_a, di_a, nf_a, er_a)
            finish(cps_b, di_b, nf_b, er_b)
            return 0

        lax.fori_loop(0, npair, pair, 0)
        if tail:
            cps = start(nchunk - 1, si_a, di_a, nf_a, er_a, ssa, sda,
                        sna, sea)
            finish(cps, di_a, nf_a, er_a)
        plsc.subcore_barrier()
        pltpu.sync_copy(agg_sh.at[pl.ds(sid * zrows, zrows)],
                        out_hbm.at[cid, pl.ds(sid * zrows, zrows)])
        if rem:
            @pl.when(sid == NS - 1)
            def _():
                pltpu.sync_copy(agg_sh.at[pl.ds(NS * zrows, rem)],
                                out_hbm.at[cid, pl.ds(NS * zrows, rem)])

    return k


def _edge_mlp(edge_in, We, be):
    E = edge_in.shape[0]
    BE = 8000
    grid = E // BE

    def body(x_ref, we_ref, be_ref, o_ref):
        h = jnp.dot(x_ref[...], we_ref[...],
                    preferred_element_type=jnp.float32) + be_ref[...]
        o_ref[...] = _gelu(h)

    return pl.pallas_call(
        body,
        grid=(grid,),
        in_specs=[
            pl.BlockSpec((BE, 4), lambda i: (i, 0)),
            pl.BlockSpec((4, 128), lambda i: (0, 0)),
            pl.BlockSpec((1, 128), lambda i: (0, 0)),
        ],
        out_specs=pl.BlockSpec((BE, 128), lambda i: (i, 0)),
        out_shape=jax.ShapeDtypeStruct((E, 128), jnp.float32),
    )(edge_in, We, be)


def _node_mlp(partials, Wn, bn):
    N = partials[0].shape[1]
    BN = 2000
    grid = N // BN

    np_ = len(partials)

    def body(*refs):
        p_refs, (wn_ref, bn_ref, o_ref) = refs[:np_], refs[np_:]
        agg = sum(p[0] + p[1] for p in p_refs)
        h = jnp.dot(agg, wn_ref[...],
                    preferred_element_type=jnp.float32) + bn_ref[...]
        o_ref[...] = _gelu(h)

    return pl.pallas_call(
        body,
        grid=(grid,),
        in_specs=[pl.BlockSpec((2, BN, 128), lambda i: (0, i, 0))
                  for _ in range(np_)] + [
            pl.BlockSpec((128, 128), lambda i: (0, 0)),
            pl.BlockSpec((1, 128), lambda i: (0, 0)),
        ],
        out_specs=pl.BlockSpec((BN, 128), lambda i: (i, 0)),
        out_shape=jax.ShapeDtypeStruct((N, 128), jnp.float32),
    )(*partials, Wn, bn)


def kernel(node_features, positions, edge_index, We, be, Wn, bn):
    N, D = node_features.shape
    E = edge_index.shape[0]
    ei = edge_index.astype(jnp.int32)
    src = ei[:, 0]
    dst = ei[:, 1]
    px = positions[:, 0]
    py = positions[:, 1]
    pz = positions[:, 2]

    edge_in = _edge_geom_kernel(E, N)(px, py, pz, src, dst).reshape(E, 4)
    edge_repr = _edge_mlp(edge_in, We, be.reshape(1, 128))

    zrows = (N // NS) // 8 * 8
    zero = jnp.zeros((zrows, 128), jnp.float32)
    partials = _msg_scatter_kernel(E, N, CB)(node_features, edge_repr,
                                             src, dst, zero)
    return _node_mlp([partials], Wn, bn.reshape(1, 128))
